# baseline (speedup 1.0000x reference)
import functools

import jax
import jax.numpy as jnp
from jax.experimental import pallas as pl
from jax.experimental.pallas import tpu as pltpu

_LANE = 128
_SUBLANE = 8
_VMEM_LIMIT = 48 * 1024 * 1024


def _ceil_to(x, m):
    return ((x + m - 1) // m) * m


def _stats_kernel(x_ref, s_ref, sx_ref):
    b = pl.program_id(1)
    t = pl.program_id(2)

    @pl.when((b == 0) & (t == 0))
    def _():
        s_ref[...] = jnp.zeros_like(s_ref)
        sx_ref[...] = jnp.zeros_like(sx_ref)

    w_blk = x_ref.shape[1]
    xs0 = x_ref[0, 0]
    acc = jax.lax.dot_general(xs0, xs0, (((1,), (1,)), ((), ())),
                              preferred_element_type=jnp.float32)
    vsum = xs0
    for w in range(1, w_blk):
        xs = x_ref[0, w]
        acc += jax.lax.dot_general(xs, xs, (((1,), (1,)), ((), ())),
                                   preferred_element_type=jnp.float32)
        vsum = vsum + xs
    s_ref[0] += acc
    sx_ref[0] += jnp.sum(vsum, axis=1, keepdims=True)


def _apply_kernel(x_ref, w_ref, s_ref, sx_ref, g_ref, b_ref, o_ref,
                  *, neg_slope, cnt, eps):
    w2 = w_ref[...]
    S = s_ref[0]
    sx = sx_ref[0]
    for c in range(1, s_ref.shape[0]):
        S = S + s_ref[c]
        sx = sx + sx_ref[c]
    mean = jax.lax.dot_general(
        w2, sx, (((1,), (0,)), ((), ())),
        preferred_element_type=jnp.float32) * (1.0 / cnt)
    t1 = jax.lax.dot_general(
        w2, S, (((1,), (0,)), ((), ())),
        preferred_element_type=jnp.float32)
    ssy = jnp.sum(t1 * w2, axis=1, keepdims=True)
    var = jnp.maximum(ssy * (1.0 / cnt) - mean * mean, 0.0)
    inv = jax.lax.rsqrt(var + eps)
    scale = g_ref[...] * inv
    shift = b_ref[...] - mean * scale

    for w in range(x_ref.shape[1]):
        z = jax.lax.dot_general(
            w2, x_ref[0, w], (((1,), (0,)), ((), ())),
            preferred_element_type=jnp.float32)
        z = z * scale + shift
        o_ref[0, w] = jnp.where(z > 0, z, neg_slope * z)


def kernel(x, weight, bias, gamma, beta):
    del bias
    eps = 1e-5
    neg_slope = 0.2

    B, Cin, H, W = x.shape
    Cout = weight.shape[0]
    M = B * H * W

    xv = x.transpose(0, 3, 1, 2).astype(jnp.float32)

    Cp = _ceil_to(Cin, _SUBLANE)
    Cop = _ceil_to(Cout, _SUBLANE)
    Hp = _ceil_to(H, _LANE)
    if (Cp, Hp) != (Cin, H):
        xv = jnp.pad(xv, ((0, 0), (0, 0), (0, Cp - Cin), (0, Hp - H)))
    w2 = weight.reshape(Cout, Cin).astype(jnp.float32)
    if (Cop, Cp) != (Cout, Cin):
        w2 = jnp.pad(w2, ((0, Cop - Cout), (0, Cp - Cin)))
    g2 = jnp.pad(gamma.astype(jnp.float32), (0, Cop - Cout))[:, None]
    b2 = jnp.pad(beta.astype(jnp.float32), (0, Cop - Cout))[:, None]

    w_blk = W
    while w_blk > 1 and (Cp * Hp * 4 * w_blk > 3 * 1024 * 1024
                         or W % w_blk != 0):
        w_blk -= 1
    nw = W // w_blk

    n_cores = 2 if B % 2 == 0 else 1
    bh = B // n_cores
    s_part, sx_part = pl.pallas_call(
        _stats_kernel,
        out_shape=(
            jax.ShapeDtypeStruct((n_cores, Cp, Cp), jnp.float32),
            jax.ShapeDtypeStruct((n_cores, Cp, 1), jnp.float32),
        ),
        grid=(n_cores, bh, nw),
        in_specs=[
            pl.BlockSpec((1, w_blk, Cp, Hp),
                         lambda c, b, t: (c * bh + b, t, 0, 0)),
        ],
        out_specs=(
            pl.BlockSpec((1, Cp, Cp), lambda c, b, t: (c, 0, 0)),
            pl.BlockSpec((1, Cp, 1), lambda c, b, t: (c, 0, 0)),
        ),
        compiler_params=pltpu.CompilerParams(
            dimension_semantics=("parallel", "arbitrary", "arbitrary"),
            vmem_limit_bytes=_VMEM_LIMIT),
    )(xv)

    o = pl.pallas_call(
        functools.partial(_apply_kernel, neg_slope=neg_slope,
                          cnt=float(M), eps=eps),
        out_shape=jax.ShapeDtypeStruct((B, W, Cop, Hp), jnp.float32),
        grid=(B, nw),
        in_specs=[
            pl.BlockSpec((1, w_blk, Cp, Hp), lambda b, t: (b, t, 0, 0)),
            pl.BlockSpec((Cop, Cp), lambda b, t: (0, 0)),
            pl.BlockSpec((n_cores, Cp, Cp), lambda b, t: (0, 0, 0)),
            pl.BlockSpec((n_cores, Cp, 1), lambda b, t: (0, 0, 0)),
            pl.BlockSpec((Cop, 1), lambda b, t: (0, 0)),
            pl.BlockSpec((Cop, 1), lambda b, t: (0, 0)),
        ],
        out_specs=pl.BlockSpec((1, w_blk, Cop, Hp), lambda b, t: (b, t, 0, 0)),
        compiler_params=pltpu.CompilerParams(
            dimension_semantics=("parallel", "arbitrary"),
            vmem_limit_bytes=_VMEM_LIMIT),
    )(xv, w2, s_part, sx_part, g2, b2)

    out = o[:, :, :Cout, :H].transpose(0, 2, 3, 1)
    return out


# speedup vs baseline: 1.2336x; 1.2336x over previous
import functools

import jax
import jax.numpy as jnp
from jax.experimental import pallas as pl
from jax.experimental.pallas import tpu as pltpu

_LANE = 128
_SUBLANE = 8
_VMEM_LIMIT = 48 * 1024 * 1024


def _ceil_to(x, m):
    return ((x + m - 1) // m) * m


def _stats_kernel(x_ref, s_ref, sx_ref):
    b = pl.program_id(1)
    t = pl.program_id(2)

    @pl.when((b == 0) & (t == 0))
    def _():
        s_ref[...] = jnp.zeros_like(s_ref)
        sx_ref[...] = jnp.zeros_like(sx_ref)

    bb, w_blk = x_ref.shape[0], x_ref.shape[1]
    xs0 = x_ref[0, 0]
    acc = jax.lax.dot_general(xs0, xs0, (((1,), (1,)), ((), ())),
                              preferred_element_type=jnp.float32)
    vsum = xs0
    for i in range(bb):
        for w in range(1 if i == 0 else 0, w_blk):
            xs = x_ref[i, w]
            acc += jax.lax.dot_general(xs, xs, (((1,), (1,)), ((), ())),
                                       preferred_element_type=jnp.float32)
            vsum = vsum + xs
    s_ref[0] += acc
    sx_ref[0] += jnp.sum(vsum, axis=1, keepdims=True)


def _apply_kernel(x_ref, w_ref, s_ref, sx_ref, g_ref, b_ref, o_ref,
                  *, neg_slope, cnt, eps):
    w2 = w_ref[...]
    S = s_ref[0]
    sx = sx_ref[0]
    for c in range(1, s_ref.shape[0]):
        S = S + s_ref[c]
        sx = sx + sx_ref[c]
    mean = jax.lax.dot_general(
        w2, sx, (((1,), (0,)), ((), ())),
        preferred_element_type=jnp.float32) * (1.0 / cnt)
    t1 = jax.lax.dot_general(
        w2, S, (((1,), (0,)), ((), ())),
        preferred_element_type=jnp.float32)
    ssy = jnp.sum(t1 * w2, axis=1, keepdims=True)
    var = jnp.maximum(ssy * (1.0 / cnt) - mean * mean, 0.0)
    inv = jax.lax.rsqrt(var + eps)
    scale = g_ref[...] * inv
    shift = b_ref[...] - mean * scale

    for w in range(x_ref.shape[1]):
        z = jax.lax.dot_general(
            w2, x_ref[0, w], (((1,), (0,)), ((), ())),
            preferred_element_type=jnp.float32)
        z = z * scale + shift
        o_ref[0, w] = jnp.where(z > 0, z, neg_slope * z)


def kernel(x, weight, bias, gamma, beta):
    del bias
    eps = 1e-5
    neg_slope = 0.2

    B, Cin, H, W = x.shape
    Cout = weight.shape[0]
    M = B * H * W

    xv = x.transpose(0, 3, 1, 2).astype(jnp.float32)

    Cp = _ceil_to(Cin, _SUBLANE)
    Cop = _ceil_to(Cout, _SUBLANE)
    Hp = _ceil_to(H, _LANE)
    if (Cp, Hp) != (Cin, H):
        xv = jnp.pad(xv, ((0, 0), (0, 0), (0, Cp - Cin), (0, Hp - H)))
    w2 = weight.reshape(Cout, Cin).astype(jnp.float32)
    if (Cop, Cp) != (Cout, Cin):
        w2 = jnp.pad(w2, ((0, Cop - Cout), (0, Cp - Cin)))
    g2 = jnp.pad(gamma.astype(jnp.float32), (0, Cop - Cout))[:, None]
    b2 = jnp.pad(beta.astype(jnp.float32), (0, Cop - Cout))[:, None]

    w_blk = W
    while w_blk > 1 and (Cp * Hp * 4 * w_blk > 6 * 1024 * 1024
                         or W % w_blk != 0):
        w_blk -= 1
    nw = W // w_blk

    n_cores = 2 if B % 2 == 0 else 1
    bh = B // n_cores
    bb = 2 if (bh % 2 == 0 and Cp * Hp * 4 * W * 2 <= 12 * 1024 * 1024) else 1
    nb = bh // bb
    s_part, sx_part = pl.pallas_call(
        _stats_kernel,
        out_shape=(
            jax.ShapeDtypeStruct((n_cores, Cp, Cp), jnp.float32),
            jax.ShapeDtypeStruct((n_cores, Cp, 1), jnp.float32),
        ),
        grid=(n_cores, nb, nw),
        in_specs=[
            pl.BlockSpec((bb, w_blk, Cp, Hp),
                         lambda c, b, t: (c * nb + b, t, 0, 0)),
        ],
        out_specs=(
            pl.BlockSpec((1, Cp, Cp), lambda c, b, t: (c, 0, 0)),
            pl.BlockSpec((1, Cp, 1), lambda c, b, t: (c, 0, 0)),
        ),
        compiler_params=pltpu.CompilerParams(
            dimension_semantics=("parallel", "arbitrary", "arbitrary"),
            vmem_limit_bytes=_VMEM_LIMIT),
    )(xv)

    o = pl.pallas_call(
        functools.partial(_apply_kernel, neg_slope=neg_slope,
                          cnt=float(M), eps=eps),
        out_shape=jax.ShapeDtypeStruct((B, W, Cop, Hp), jnp.float32),
        grid=(B, nw),
        in_specs=[
            pl.BlockSpec((1, w_blk, Cp, Hp), lambda b, t: (b, t, 0, 0)),
            pl.BlockSpec((Cop, Cp), lambda b, t: (0, 0)),
            pl.BlockSpec((n_cores, Cp, Cp), lambda b, t: (0, 0, 0)),
            pl.BlockSpec((n_cores, Cp, 1), lambda b, t: (0, 0, 0)),
            pl.BlockSpec((Cop, 1), lambda b, t: (0, 0)),
            pl.BlockSpec((Cop, 1), lambda b, t: (0, 0)),
        ],
        out_specs=pl.BlockSpec((1, w_blk, Cop, Hp), lambda b, t: (b, t, 0, 0)),
        compiler_params=pltpu.CompilerParams(
            dimension_semantics=("parallel", "arbitrary"),
            vmem_limit_bytes=_VMEM_LIMIT),
    )(xv, w2, s_part, sx_part, g2, b2)

    out = o[:, :, :Cout, :H].transpose(0, 2, 3, 1)
    return out


# speedup vs baseline: 1.2505x; 1.0137x over previous
import functools

import jax
import jax.numpy as jnp
from jax.experimental import pallas as pl
from jax.experimental.pallas import tpu as pltpu

_LANE = 128
_SUBLANE = 8
_VMEM_LIMIT = 48 * 1024 * 1024


def _ceil_to(x, m):
    return ((x + m - 1) // m) * m


def _stats_kernel(x_ref, s_ref, sx_ref):
    b = pl.program_id(1)
    t = pl.program_id(2)

    @pl.when((b == 0) & (t == 0))
    def _():
        s_ref[...] = jnp.zeros_like(s_ref)
        sx_ref[...] = jnp.zeros_like(sx_ref)

    bb, w_blk = x_ref.shape[0], x_ref.shape[1]
    xs0 = x_ref[0, 0]
    acc = jax.lax.dot_general(xs0, xs0, (((1,), (1,)), ((), ())),
                              preferred_element_type=jnp.float32)
    vsum = xs0
    for i in range(bb):
        for w in range(1 if i == 0 else 0, w_blk):
            xs = x_ref[i, w]
            acc += jax.lax.dot_general(xs, xs, (((1,), (1,)), ((), ())),
                                       preferred_element_type=jnp.float32)
            vsum = vsum + xs
    s_ref[0] += acc
    sx_ref[0] += jnp.sum(vsum, axis=1, keepdims=True)


def _apply_kernel(x_ref, w_ref, s_ref, sx_ref, g_ref, b_ref, o_ref,
                  *, neg_slope, cnt, eps):
    w2 = w_ref[...]
    S = s_ref[0]
    sx = sx_ref[0]
    for c in range(1, s_ref.shape[0]):
        S = S + s_ref[c]
        sx = sx + sx_ref[c]
    mean = jax.lax.dot_general(
        w2, sx, (((1,), (0,)), ((), ())),
        preferred_element_type=jnp.float32) * (1.0 / cnt)
    t1 = jax.lax.dot_general(
        w2, S, (((1,), (0,)), ((), ())),
        preferred_element_type=jnp.float32)
    ssy = jnp.sum(t1 * w2, axis=1, keepdims=True)
    var = jnp.maximum(ssy * (1.0 / cnt) - mean * mean, 0.0)
    inv = jax.lax.rsqrt(var + eps)
    scale = g_ref[...] * inv
    shift = b_ref[...] - mean * scale

    for i in range(x_ref.shape[0]):
        for w in range(x_ref.shape[1]):
            z = jax.lax.dot_general(
                w2, x_ref[i, w], (((1,), (0,)), ((), ())),
                preferred_element_type=jnp.float32)
            z = z * scale + shift
            o_ref[i, w] = jnp.where(z > 0, z, neg_slope * z)


def kernel(x, weight, bias, gamma, beta):
    del bias
    eps = 1e-5
    neg_slope = 0.2

    B, Cin, H, W = x.shape
    Cout = weight.shape[0]
    M = B * H * W

    xv = x.transpose(0, 3, 1, 2).astype(jnp.float32)

    Cp = _ceil_to(Cin, _SUBLANE)
    Cop = _ceil_to(Cout, _SUBLANE)
    Hp = _ceil_to(H, _LANE)
    if (Cp, Hp) != (Cin, H):
        xv = jnp.pad(xv, ((0, 0), (0, 0), (0, Cp - Cin), (0, Hp - H)))
    w2 = weight.reshape(Cout, Cin).astype(jnp.float32)
    if (Cop, Cp) != (Cout, Cin):
        w2 = jnp.pad(w2, ((0, Cop - Cout), (0, Cp - Cin)))
    g2 = jnp.pad(gamma.astype(jnp.float32), (0, Cop - Cout))[:, None]
    b2 = jnp.pad(beta.astype(jnp.float32), (0, Cop - Cout))[:, None]

    w_blk = W
    while w_blk > 1 and (Cp * Hp * 4 * w_blk > 6 * 1024 * 1024
                         or W % w_blk != 0):
        w_blk -= 1
    nw = W // w_blk

    n_cores = 2 if B % 2 == 0 else 1
    bh = B // n_cores
    bb = 2 if (bh % 2 == 0 and Cp * Hp * 4 * W * 2 <= 12 * 1024 * 1024) else 1
    nb = bh // bb
    s_part, sx_part = pl.pallas_call(
        _stats_kernel,
        out_shape=(
            jax.ShapeDtypeStruct((n_cores, Cp, Cp), jnp.float32),
            jax.ShapeDtypeStruct((n_cores, Cp, 1), jnp.float32),
        ),
        grid=(n_cores, nb, nw),
        in_specs=[
            pl.BlockSpec((bb, w_blk, Cp, Hp),
                         lambda c, b, t: (c * nb + b, t, 0, 0)),
        ],
        out_specs=(
            pl.BlockSpec((1, Cp, Cp), lambda c, b, t: (c, 0, 0)),
            pl.BlockSpec((1, Cp, 1), lambda c, b, t: (c, 0, 0)),
        ),
        compiler_params=pltpu.CompilerParams(
            dimension_semantics=("parallel", "arbitrary", "arbitrary"),
            vmem_limit_bytes=_VMEM_LIMIT),
    )(xv)

    bb2 = 2 if (B % 2 == 0 and Cp * Hp * 4 * W * 4 <= 44 * 1024 * 1024) else 1
    o = pl.pallas_call(
        functools.partial(_apply_kernel, neg_slope=neg_slope,
                          cnt=float(M), eps=eps),
        out_shape=jax.ShapeDtypeStruct((B, W, Cop, Hp), jnp.float32),
        grid=(B // bb2, nw),
        in_specs=[
            pl.BlockSpec((bb2, w_blk, Cp, Hp), lambda b, t: (b, t, 0, 0)),
            pl.BlockSpec((Cop, Cp), lambda b, t: (0, 0)),
            pl.BlockSpec((n_cores, Cp, Cp), lambda b, t: (0, 0, 0)),
            pl.BlockSpec((n_cores, Cp, 1), lambda b, t: (0, 0, 0)),
            pl.BlockSpec((Cop, 1), lambda b, t: (0, 0)),
            pl.BlockSpec((Cop, 1), lambda b, t: (0, 0)),
        ],
        out_specs=pl.BlockSpec((bb2, w_blk, Cop, Hp), lambda b, t: (b, t, 0, 0)),
        compiler_params=pltpu.CompilerParams(
            dimension_semantics=("parallel", "arbitrary"),
            vmem_limit_bytes=_VMEM_LIMIT),
    )(xv, w2, s_part, sx_part, g2, b2)

    out = o[:, :, :Cout, :H].transpose(0, 2, 3, 1)
    return out
